# Initial kernel scaffold; baseline (speedup 1.0000x reference)
import jax
import jax.numpy as jnp
from jax.experimental import pallas as pl


def kernel(x, edge_index, W0, b0, g0, be0, W1, b1, g1, be1, W2, b2):
    raise NotImplementedError("write your pallas kernel here")


# speedup vs baseline: 10.8264x; 10.8264x over previous
import functools

import jax
import jax.numpy as jnp
from jax import lax
from jax.experimental import pallas as pl
from jax.experimental.pallas import tpu as pltpu
from jax.experimental.pallas import tpu_sc as plsc

N = 10000
NPAD = 10240
E = 320000
D = 128

NC = 2
NS = 16
NW = NC * NS
EPW = E // NW
CH = 80
NCH = EPW // CH
SLAB = NPAD // NS

_mesh = functools.partial(
    plsc.VectorSubcoreMesh, core_axis_name="c", subcore_axis_name="s")


def _sc_deg_body(edge3, z1_hbm, out_hbm, acc, idxv, onesv, tmp):
  c = lax.axis_index("c")
  s = lax.axis_index("s")
  wid = s * NC + c
  pltpu.sync_copy(z1_hbm.at[pl.ds(s * SLAB, SLAB)],
                  acc.at[pl.ds(s * SLAB, SLAB)])
  for k in range(CH // 16):
    onesv[0, pl.ds(k * 16, 16)] = jnp.ones((16,), jnp.float32)
  plsc.subcore_barrier()

  @pl.loop(0, NCH)
  def _(i):
    cb = wid * NCH + i
    pltpu.sync_copy(edge3.at[1, cb], idxv.at[0])
    pltpu.sync_copy(onesv.at[0], acc.at[idxv.at[0]], add=True)

  plsc.subcore_barrier()
  pltpu.sync_copy(acc.at[pl.ds(s * SLAB, SLAB)], tmp)
  pltpu.sync_copy(tmp, out_hbm.at[c, pl.ds(s * SLAB, SLAB)])


def _sc_deg(edge3, z1):
  return pl.kernel(
      _sc_deg_body,
      out_type=jax.ShapeDtypeStruct((NC, NPAD), jnp.float32),
      mesh=_mesh(),
      scratch_types=[
          pltpu.VMEM_SHARED((NPAD,), jnp.float32),
          pltpu.VMEM((2, CH), jnp.int32),
          pltpu.VMEM((1, CH), jnp.float32),
          pltpu.VMEM((SLAB,), jnp.float32),
      ],
  )(edge3, z1)


def _sc_agg_body(edge3, p_hbm, z2_hbm, out_hbm, acc, idxv, rows, tmp):
  c = lax.axis_index("c")
  s = lax.axis_index("s")
  wid = s * NC + c
  pltpu.sync_copy(z2_hbm.at[pl.ds(s * SLAB, SLAB)],
                  acc.at[pl.ds(s * SLAB, SLAB)])
  plsc.subcore_barrier()

  @pl.loop(0, NCH)
  def _(i):
    cb = wid * NCH + i
    pltpu.sync_copy(edge3.at[0, cb], idxv.at[0])
    pltpu.sync_copy(edge3.at[1, cb], idxv.at[1])
    pltpu.sync_copy(p_hbm.at[idxv.at[0]], rows)
    pltpu.sync_copy(rows, acc.at[idxv.at[1]], add=True)

  plsc.subcore_barrier()

  @pl.loop(0, SLAB // CH)
  def _(k):
    r0 = s * SLAB + k * CH
    pltpu.sync_copy(acc.at[pl.ds(r0, CH)], tmp)
    pltpu.sync_copy(tmp, out_hbm.at[c, pl.ds(r0, CH)])


def _sc_agg(edge3, p, z2):
  return pl.kernel(
      _sc_agg_body,
      out_type=jax.ShapeDtypeStruct((NC, NPAD, D), jnp.float32),
      mesh=_mesh(),
      scratch_types=[
          pltpu.VMEM_SHARED((NPAD, D), jnp.float32),
          pltpu.VMEM((2, CH), jnp.int32),
          pltpu.VMEM((CH, D), jnp.float32),
          pltpu.VMEM((CH, D), jnp.float32),
      ],
  )(edge3, p, z2)


BR = 1000


def _dinv(degT):
  return lax.rsqrt(degT[:, 0:1] + degT[:, 1:2] + 1.0)


def _tc_pre_body(x_ref, degT_ref, p_ref):
  p_ref[...] = x_ref[...] * _dinv(degT_ref[...])


def _tc_pre(x, degT):
  return pl.pallas_call(
      _tc_pre_body,
      grid=(N // BR,),
      in_specs=[
          pl.BlockSpec((BR, D), lambda i: (i, 0)),
          pl.BlockSpec((BR, 2), lambda i: (i, 0)),
      ],
      out_specs=pl.BlockSpec((BR, D), lambda i: (i, 0)),
      out_shape=jax.ShapeDtypeStruct((N, D), jnp.float32),
  )(x, degT)


def _agg_block(parts_ref, h_ref, degT_ref):
  dinv = _dinv(degT_ref[...])
  s = parts_ref[0] + parts_ref[1]
  return dinv * s + (dinv * dinv) * h_ref[...], dinv


def _tc_layer_body(parts_ref, h_ref, degT_ref, w_ref, b_ref, g_ref, be_ref,
                   h1_ref, p1_ref):
  agg, dinv = _agg_block(parts_ref, h_ref, degT_ref)
  t = jnp.dot(agg, w_ref[...], preferred_element_type=jnp.float32) + b_ref[...]
  mu = jnp.mean(t, axis=-1, keepdims=True)
  var = jnp.mean(jnp.square(t - mu), axis=-1, keepdims=True)
  tn = (t - mu) * lax.rsqrt(var + 1e-5) * g_ref[...] + be_ref[...]
  h1 = jnp.maximum(tn, 0.0)
  h1_ref[...] = h1
  p1_ref[...] = dinv * h1


def _tc_layer(parts, h, degT, w, b, g, be):
  return pl.pallas_call(
      _tc_layer_body,
      grid=(N // BR,),
      in_specs=[
          pl.BlockSpec((NC, BR, D), lambda i: (0, i, 0)),
          pl.BlockSpec((BR, D), lambda i: (i, 0)),
          pl.BlockSpec((BR, 2), lambda i: (i, 0)),
          pl.BlockSpec((D, D), lambda i: (0, 0)),
          pl.BlockSpec((1, D), lambda i: (0, 0)),
          pl.BlockSpec((1, D), lambda i: (0, 0)),
          pl.BlockSpec((1, D), lambda i: (0, 0)),
      ],
      out_specs=[
          pl.BlockSpec((BR, D), lambda i: (i, 0)),
          pl.BlockSpec((BR, D), lambda i: (i, 0)),
      ],
      out_shape=[
          jax.ShapeDtypeStruct((N, D), jnp.float32),
          jax.ShapeDtypeStruct((N, D), jnp.float32),
      ],
  )(parts, h, degT, w, b, g, be)


DOUT = 40


def _tc_final_body(parts_ref, h_ref, degT_ref, w_ref, b_ref, o_ref):
  agg, _ = _agg_block(parts_ref, h_ref, degT_ref)
  t = jnp.dot(agg, w_ref[...], preferred_element_type=jnp.float32) + b_ref[...]
  m = jnp.max(t, axis=-1, keepdims=True)
  lse = jnp.log(jnp.sum(jnp.exp(t - m), axis=-1, keepdims=True)) + m
  o_ref[...] = t - lse


def _tc_final(parts, h, degT, w, b):
  return pl.pallas_call(
      _tc_final_body,
      grid=(N // BR,),
      in_specs=[
          pl.BlockSpec((NC, BR, D), lambda i: (0, i, 0)),
          pl.BlockSpec((BR, D), lambda i: (i, 0)),
          pl.BlockSpec((BR, 2), lambda i: (i, 0)),
          pl.BlockSpec((D, DOUT), lambda i: (0, 0)),
          pl.BlockSpec((1, DOUT), lambda i: (0, 0)),
      ],
      out_specs=pl.BlockSpec((BR, DOUT), lambda i: (i, 0)),
      out_shape=jax.ShapeDtypeStruct((N, DOUT), jnp.float32),
  )(parts, h, degT, w, b)


def kernel(x, edge_index, W0, b0, g0, be0, W1, b1, g1, be1, W2, b2):
  edge3 = edge_index.reshape(2, E // CH, CH)
  z1 = jnp.zeros((NPAD,), jnp.float32)
  z2 = jnp.zeros((NPAD, D), jnp.float32)

  deg_parts = _sc_deg(edge3, z1)
  degT = deg_parts[:, :N].T

  p0 = _tc_pre(x, degT)
  parts0 = _sc_agg(edge3, p0, z2)
  h1, p1 = _tc_layer(parts0, x, degT, W0, b0.reshape(1, D),
                     g0.reshape(1, D), be0.reshape(1, D))
  parts1 = _sc_agg(edge3, p1, z2)
  h2, p2 = _tc_layer(parts1, h1, degT, W1, b1.reshape(1, D),
                     g1.reshape(1, D), be1.reshape(1, D))
  parts2 = _sc_agg(edge3, p2, z2)
  return _tc_final(parts2, h2, degT, W2, b2.reshape(1, DOUT))
